# Initial kernel scaffold; baseline (speedup 1.0000x reference)
import jax
import jax.numpy as jnp
from jax.experimental import pallas as pl


def kernel(x, memory, momentum_buffer, Wk, Wv, Wq, Wo, gk, bk, gv, bv, gq, bq, lr_scale, momentum_scale, Wg, bg):
    raise NotImplementedError("write your pallas kernel here")


# speedup vs baseline: 7.0388x; 7.0388x over previous
import functools

import jax
import jax.numpy as jnp
from jax.experimental import pallas as pl
from jax.experimental.pallas import tpu as pltpu

DIM = 1024
HD = 64
NH = 8
BASE_LR = 0.1
BASE_MOM = 0.9
EPS = 1e-6
LN_EPS = 1e-5
CHUNK = 64


_PREC = jax.lax.Precision.HIGHEST


def _dot(a, b):
    return jnp.dot(a, b, preferred_element_type=jnp.float32,
                   precision=_PREC)


def _dot_nt(a, b):
    return jax.lax.dot_general(a, b, (((1,), (1,)), ((), ())),
                               preferred_element_type=jnp.float32,
                               precision=_PREC)


def _dot_tn(a, b):
    return jax.lax.dot_general(a, b, (((0,), (0,)), ((), ())),
                               preferred_element_type=jnp.float32,
                               precision=_PREC)


def _dot_bf(a, b):
    return jnp.dot(a.astype(jnp.bfloat16), b.astype(jnp.bfloat16),
                   preferred_element_type=jnp.float32)


def _proj_body(x_ref, wk_ref, wv_ref, wq_ref, ones_ref, g_ref, b_ref,
               kn_ref, v_ref, q_ref):
    xb = x_ref[...]
    ones = ones_ref[...]
    inv64 = 1.0 / HD

    def ln(t, off):
        mean = _dot(t, ones) * inv64
        msq = _dot(t * t, ones) * inv64
        var = msq - mean * mean
        g = g_ref[0:1, off:off + NH * HD]
        b = b_ref[0:1, off:off + NH * HD]
        return (t - mean) * jax.lax.rsqrt(var + LN_EPS) * g + b

    k = ln(_dot_bf(xb, wk_ref[...]), 0)
    ss = _dot(k * k, ones)
    kn_ref[...] = k * (1.0 / (jnp.sqrt(ss) + EPS))
    v_ref[...] = ln(_dot_bf(xb, wv_ref[...]), NH * HD)
    q_ref[...] = ln(_dot_bf(xb, wq_ref[...]), 2 * NH * HD)


def _scan_body(nc, batch, kn_ref, v_ref, q_ref, p_ref, qq_ref, col_ref,
               mem_ref, mbuf_ref, out_ref, memo_ref, mbufo_ref):
    c = pl.program_id(1)

    @pl.when(c == 0)
    def _():
        memo_ref[...] = mem_ref[...]
        mbufo_ref[...] = mbuf_ref[...]

    P = p_ref[0]
    Qq = qq_ref[0]
    g2 = col_ref[0, :, 0:1]
    cP = col_ref[0, :, 1:2]
    cQ = col_ref[0, :, 2:3]
    dm = col_ref[0, :, 3:4]
    aGC = col_ref[0, :, 4:5]
    momC = col_ref[0, :, 5:6]

    for b in range(batch):
        mT = memo_ref[b, 0]
        bT = mbufo_ref[b, 0]
        Kn = kn_ref[0, b]
        V = v_ref[0, b]
        Qh = q_ref[0, b]

        Smat = _dot_nt(Kn, Kn)
        B0 = _dot(Kn, mT) - g2 * _dot(Kn, bT) + _dot(Qq * Smat, V)
        N = -(P * Smat)
        U = B0
        for i in range(6):
            U = U + _dot(N, U)
            if i < 5:
                N = _dot(N, N)

        Sq = _dot_nt(Qh, Kn)
        out_ref[0, b] = (_dot(Qh, mT) - g2 * _dot(Qh, bT)
                         - _dot(P * Sq, U) + _dot(Qq * Sq, V))

        W1 = cP * U - cQ * V
        memo_ref[b, 0] = mT - aGC * bT - _dot_tn(Kn, W1)
        mbufo_ref[b, 0] = momC * bT + _dot_tn(Kn, dm * (U - V))


def _out_body(o_ref, x_ref, wo_ref, wg_ref, bg_ref, y_ref):
    gate = jax.nn.sigmoid(_dot_bf(x_ref[...], wg_ref[...]) + bg_ref[0:1, :])
    y_ref[...] = gate * _dot_bf(o_ref[...], wo_ref[...])


def kernel(x, memory, momentum_buffer, Wk, Wv, Wq, Wo, gk, bk, gv, bv, gq,
           bq, lr_scale, momentum_scale, Wg, bg):
    B, S, _ = x.shape
    C = CHUNK
    NC = S // C
    HDN = NH * HD
    R = 256
    xr = x.reshape(B * S, DIM)

    ones_blk = jnp.kron(jnp.eye(NH, dtype=jnp.float32),
                        jnp.ones((HD, HD), jnp.float32))
    gcat = jnp.concatenate([jnp.tile(gk, NH), jnp.tile(gv, NH),
                            jnp.tile(gq, NH)])[None, :].repeat(8, 0)
    bcat = jnp.concatenate([jnp.tile(bk, NH), jnp.tile(bv, NH),
                            jnp.tile(bq, NH)])[None, :].repeat(8, 0)

    lr = jax.nn.sigmoid(lr_scale) * BASE_LR * 2.0
    mom = jax.nn.sigmoid(momentum_scale) * BASE_MOM * 2.0
    a = lr * mom
    pw = mom[:, None] ** jnp.arange(C + 1, dtype=jnp.float32)
    Gtab = jnp.concatenate(
        [jnp.zeros((NH, 1), jnp.float32), jnp.cumsum(pw[:, :C], axis=1)], 1)
    ii = jnp.arange(C)[:, None]
    rr = jnp.arange(C)[None, :]
    low = (ii > rr)
    gidx = jnp.clip(ii - rr - 1, 0, C)
    Gv = Gtab[:, gidx]
    Pm = jnp.where(low[None], a[:, None, None] * Gv + 1.0 + lr[:, None, None],
                   0.0)
    Qm = jnp.where(low[None], a[:, None, None] * Gv + lr[:, None, None], 0.0)
    g2 = a[:, None] * Gtab[:, :C]
    gC = Gtab[:, C - 1 - jnp.arange(C)]
    cP = a[:, None] * gC + 1.0 + lr[:, None]
    cQ = a[:, None] * gC + lr[:, None]
    dm = pw[:, C - 1 - jnp.arange(C)]
    aGC = (a * Gtab[:, C])[:, None].repeat(C, 1)
    momC = pw[:, C][:, None].repeat(C, 1)
    cols = jnp.stack([g2, cP, cQ, dm, aGC, momC], axis=-1)
    cols = jnp.concatenate(
        [cols, jnp.zeros((NH, C, 128 - 6), jnp.float32)], -1)

    grid1 = (B * S // R,)
    kn, v, q = pl.pallas_call(
        _proj_body,
        grid=grid1,
        in_specs=[
            pl.BlockSpec((R, DIM), lambda i: (i, 0)),
            pl.BlockSpec((DIM, HDN), lambda i: (0, 0)),
            pl.BlockSpec((DIM, HDN), lambda i: (0, 0)),
            pl.BlockSpec((DIM, HDN), lambda i: (0, 0)),
            pl.BlockSpec((HDN, HDN), lambda i: (0, 0)),
            pl.BlockSpec((8, 3 * HDN), lambda i: (0, 0)),
            pl.BlockSpec((8, 3 * HDN), lambda i: (0, 0)),
        ],
        out_specs=[
            pl.BlockSpec((R, HDN), lambda i: (i, 0)),
            pl.BlockSpec((R, HDN), lambda i: (i, 0)),
            pl.BlockSpec((R, HDN), lambda i: (i, 0)),
        ],
        out_shape=[jax.ShapeDtypeStruct((B * S, HDN), jnp.float32)] * 3,
        compiler_params=pltpu.CompilerParams(
            dimension_semantics=("parallel",)),
    )(xr, Wk, Wv, Wq, ones_blk, gcat, bcat)

    kn4 = kn.reshape(B, S, NH, HD).transpose(2, 0, 1, 3)
    v4 = v.reshape(B, S, NH, HD).transpose(2, 0, 1, 3)
    q4 = q.reshape(B, S, NH, HD).transpose(2, 0, 1, 3)
    memT = memory.transpose(0, 1, 3, 2)
    mbufT = momentum_buffer.transpose(0, 1, 3, 2)

    grid2 = (NH, NC)
    seq_spec = pl.BlockSpec((1, B, C, HD), lambda h, c: (h, 0, c, 0))
    st_spec = pl.BlockSpec((B, 1, HD, HD), lambda h, c: (0, h, 0, 0))
    out_scan, memT_f, mbufT_f = pl.pallas_call(
        functools.partial(_scan_body, NC, B),
        grid=grid2,
        in_specs=[
            seq_spec, seq_spec, seq_spec,
            pl.BlockSpec((1, C, C), lambda h, c: (h, 0, 0)),
            pl.BlockSpec((1, C, C), lambda h, c: (h, 0, 0)),
            pl.BlockSpec((1, C, 128), lambda h, c: (h, 0, 0)),
            st_spec, st_spec,
        ],
        out_specs=[seq_spec, st_spec, st_spec],
        out_shape=[
            jax.ShapeDtypeStruct((NH, B, S, HD), jnp.float32),
            jax.ShapeDtypeStruct((B, NH, HD, HD), jnp.float32),
            jax.ShapeDtypeStruct((B, NH, HD, HD), jnp.float32),
        ],
        compiler_params=pltpu.CompilerParams(
            dimension_semantics=("parallel", "arbitrary")),
    )(kn4, v4, q4, Pm, Qm, cols, memT, mbufT)

    bgr = bg[None, :].repeat(8, 0)
    y = pl.pallas_call(
        _out_body,
        grid=grid1,
        in_specs=[
            pl.BlockSpec((R, HDN), lambda i: (i, 0)),
            pl.BlockSpec((R, DIM), lambda i: (i, 0)),
            pl.BlockSpec((HDN, DIM), lambda i: (0, 0)),
            pl.BlockSpec((DIM, DIM), lambda i: (0, 0)),
            pl.BlockSpec((8, DIM), lambda i: (0, 0)),
        ],
        out_specs=pl.BlockSpec((R, DIM), lambda i: (i, 0)),
        out_shape=jax.ShapeDtypeStruct((B * S, DIM), jnp.float32),
        compiler_params=pltpu.CompilerParams(
            dimension_semantics=("parallel",)),
    )(out_scan.transpose(1, 2, 0, 3).reshape(B * S, HDN), xr, Wo, Wg, bgr)

    return (y.reshape(B, S, DIM),
            memT_f.transpose(0, 1, 3, 2),
            mbufT_f.transpose(0, 1, 3, 2))


# speedup vs baseline: 7.8140x; 1.1101x over previous
import functools

import jax
import jax.numpy as jnp
from jax.experimental import pallas as pl
from jax.experimental.pallas import tpu as pltpu

DIM = 1024
HD = 64
NH = 8
BASE_LR = 0.1
BASE_MOM = 0.9
EPS = 1e-6
LN_EPS = 1e-5
CHUNK = 64


def _dot(a, b, prec=jax.lax.Precision.HIGHEST):
    return jnp.dot(a, b, preferred_element_type=jnp.float32, precision=prec)


def _dot_s(a, b):
    return jnp.dot(a, b, preferred_element_type=jnp.float32,
                   precision=jax.lax.Precision.HIGHEST)


def _dot_nt(a, b):
    return jax.lax.dot_general(a, b, (((1,), (1,)), ((), ())),
                               preferred_element_type=jnp.float32,
                               precision=jax.lax.Precision.HIGHEST)


def _dot_tn(a, b):
    return jax.lax.dot_general(a, b, (((0,), (0,)), ((), ())),
                               preferred_element_type=jnp.float32,
                               precision=jax.lax.Precision.HIGHEST)


def _dot_bf(a, b):
    return jnp.dot(a.astype(jnp.bfloat16), b.astype(jnp.bfloat16),
                   preferred_element_type=jnp.float32)


def _proj_body(x_ref, wk_ref, wv_ref, wq_ref, ones_ref, g_ref, b_ref,
               kn_ref, v_ref, q_ref):
    xb = x_ref[...]
    ones = ones_ref[...]
    inv64 = 1.0 / HD

    def ln(t, off):
        mean = _dot(t, ones) * inv64
        msq = _dot(t * t, ones) * inv64
        var = msq - mean * mean
        g = g_ref[0:1, off:off + NH * HD]
        b = b_ref[0:1, off:off + NH * HD]
        return (t - mean) * jax.lax.rsqrt(var + LN_EPS) * g + b

    k = ln(_dot_bf(xb, wk_ref[...]), 0)
    ss = _dot(k * k, ones)
    kn_ref[...] = k * (1.0 / (jnp.sqrt(ss) + EPS))
    v_ref[...] = ln(_dot_bf(xb, wv_ref[...]), NH * HD)
    q_ref[...] = ln(_dot_bf(xb, wq_ref[...]), 2 * NH * HD)


def _scan_body(nc, kn_ref, v_ref, q_ref, p_ref, qq_ref, col_ref,
               mem_ref, mbuf_ref, out_ref, memo_ref, mbufo_ref):
    c = pl.program_id(1)

    @pl.when(c == 0)
    def _():
        memo_ref[...] = mem_ref[...]
        mbufo_ref[...] = mbuf_ref[...]

    kn_all = kn_ref[0]
    v_all = v_ref[0]
    q_all = q_ref[0]

    outs = []
    for h in range(NH):
        sl = slice(h * HD, (h + 1) * HD)
        Kn = kn_all[:, sl]
        V = v_all[:, sl]
        Qh = q_all[:, sl]
        P = p_ref[h]
        Qq = qq_ref[h]
        g2 = col_ref[h, :, 0:1]
        cP = col_ref[h, :, 1:2]
        cQ = col_ref[h, :, 2:3]
        dm = col_ref[h, :, 3:4]
        aGC = col_ref[h, :, 4:5]
        momC = col_ref[h, :, 5:6]

        mT = memo_ref[0, h]
        bT = mbufo_ref[0, h]

        Smat = _dot_nt(Kn, Kn)
        B0 = _dot_s(Kn, mT) - g2 * _dot_s(Kn, bT) + _dot_s(Qq * Smat, V)
        N = -(P * Smat)
        U = B0
        for i in range(6):
            U = U + _dot_s(N, U)
            if i < 5:
                N = _dot_s(N, N)

        Sq = _dot_nt(Qh, Kn)
        outs.append(_dot_s(Qh, mT) - g2 * _dot_s(Qh, bT)
                    - _dot_s(P * Sq, U) + _dot_s(Qq * Sq, V))

        W1 = cP * U - cQ * V
        memo_ref[0, h] = mT - aGC * bT - _dot_tn(Kn, W1)
        mbufo_ref[0, h] = momC * bT + _dot_tn(Kn, dm * (U - V))

    out_ref[0] = jnp.concatenate(outs, axis=-1)


def _out_body(o_ref, x_ref, wo_ref, wg_ref, bg_ref, y_ref):
    gate = jax.nn.sigmoid(_dot_bf(x_ref[...], wg_ref[...]) + bg_ref[0:1, :])
    y_ref[...] = gate * _dot_bf(o_ref[...], wo_ref[...])


def kernel(x, memory, momentum_buffer, Wk, Wv, Wq, Wo, gk, bk, gv, bv, gq,
           bq, lr_scale, momentum_scale, Wg, bg):
    B, S, _ = x.shape
    C = CHUNK
    NC = S // C
    HDN = NH * HD
    R = 256
    xr = x.reshape(B * S, DIM)

    ones_blk = jnp.kron(jnp.eye(NH, dtype=jnp.float32),
                        jnp.ones((HD, HD), jnp.float32))
    gcat = jnp.concatenate([jnp.tile(gk, NH), jnp.tile(gv, NH),
                            jnp.tile(gq, NH)])[None, :].repeat(8, 0)
    bcat = jnp.concatenate([jnp.tile(bk, NH), jnp.tile(bv, NH),
                            jnp.tile(bq, NH)])[None, :].repeat(8, 0)

    lr = jax.nn.sigmoid(lr_scale) * BASE_LR * 2.0
    mom = jax.nn.sigmoid(momentum_scale) * BASE_MOM * 2.0
    a = lr * mom
    pw = mom[:, None] ** jnp.arange(C + 1, dtype=jnp.float32)
    Gtab = jnp.concatenate(
        [jnp.zeros((NH, 1), jnp.float32), jnp.cumsum(pw[:, :C], axis=1)], 1)
    ii = jnp.arange(C)[:, None]
    rr = jnp.arange(C)[None, :]
    low = (ii > rr)
    gidx = jnp.clip(ii - rr - 1, 0, C)
    Gv = Gtab[:, gidx]
    Pm = jnp.where(low[None], a[:, None, None] * Gv + 1.0 + lr[:, None, None],
                   0.0)
    Qm = jnp.where(low[None], a[:, None, None] * Gv + lr[:, None, None], 0.0)
    g2 = a[:, None] * Gtab[:, :C]
    gC = Gtab[:, C - 1 - jnp.arange(C)]
    cP = a[:, None] * gC + 1.0 + lr[:, None]
    cQ = a[:, None] * gC + lr[:, None]
    dm = pw[:, C - 1 - jnp.arange(C)]
    aGC = (a * Gtab[:, C])[:, None].repeat(C, 1)
    momC = pw[:, C][:, None].repeat(C, 1)
    cols = jnp.stack([g2, cP, cQ, dm, aGC, momC], axis=-1)
    cols = jnp.concatenate(
        [cols, jnp.zeros((NH, C, 128 - 6), jnp.float32)], -1)

    grid1 = (B * S // R,)
    kn, v, q = pl.pallas_call(
        _proj_body,
        grid=grid1,
        in_specs=[
            pl.BlockSpec((R, DIM), lambda i: (i, 0)),
            pl.BlockSpec((DIM, HDN), lambda i: (0, 0)),
            pl.BlockSpec((DIM, HDN), lambda i: (0, 0)),
            pl.BlockSpec((DIM, HDN), lambda i: (0, 0)),
            pl.BlockSpec((HDN, HDN), lambda i: (0, 0)),
            pl.BlockSpec((8, 3 * HDN), lambda i: (0, 0)),
            pl.BlockSpec((8, 3 * HDN), lambda i: (0, 0)),
        ],
        out_specs=[
            pl.BlockSpec((R, HDN), lambda i: (i, 0)),
            pl.BlockSpec((R, HDN), lambda i: (i, 0)),
            pl.BlockSpec((R, HDN), lambda i: (i, 0)),
        ],
        out_shape=[jax.ShapeDtypeStruct((B * S, HDN), jnp.float32)] * 3,
        compiler_params=pltpu.CompilerParams(
            dimension_semantics=("parallel",)),
    )(xr, Wk, Wv, Wq, ones_blk, gcat, bcat)

    kn3 = kn.reshape(B, S, HDN)
    v3 = v.reshape(B, S, HDN)
    q3 = q.reshape(B, S, HDN)
    memT = memory.transpose(0, 1, 3, 2)
    mbufT = momentum_buffer.transpose(0, 1, 3, 2)

    grid2 = (B, NC)
    seq_spec = pl.BlockSpec((1, C, HDN), lambda b, c: (b, c, 0))
    st_spec = pl.BlockSpec((1, NH, HD, HD), lambda b, c: (b, 0, 0, 0))
    cst = lambda shape: pl.BlockSpec(shape, lambda b, c: (0,) * len(shape))
    out_scan, memT_f, mbufT_f = pl.pallas_call(
        functools.partial(_scan_body, NC),
        grid=grid2,
        in_specs=[
            seq_spec, seq_spec, seq_spec,
            cst((NH, C, C)),
            cst((NH, C, C)),
            cst((NH, C, 128)),
            st_spec, st_spec,
        ],
        out_specs=[seq_spec, st_spec, st_spec],
        out_shape=[
            jax.ShapeDtypeStruct((B, S, HDN), jnp.float32),
            jax.ShapeDtypeStruct((B, NH, HD, HD), jnp.float32),
            jax.ShapeDtypeStruct((B, NH, HD, HD), jnp.float32),
        ],
        compiler_params=pltpu.CompilerParams(
            dimension_semantics=("parallel", "arbitrary")),
    )(kn3, v3, q3, Pm, Qm, cols, memT, mbufT)

    bgr = bg[None, :].repeat(8, 0)
    y = pl.pallas_call(
        _out_body,
        grid=grid1,
        in_specs=[
            pl.BlockSpec((R, HDN), lambda i: (i, 0)),
            pl.BlockSpec((R, DIM), lambda i: (i, 0)),
            pl.BlockSpec((HDN, DIM), lambda i: (0, 0)),
            pl.BlockSpec((DIM, DIM), lambda i: (0, 0)),
            pl.BlockSpec((8, DIM), lambda i: (0, 0)),
        ],
        out_specs=pl.BlockSpec((R, DIM), lambda i: (i, 0)),
        out_shape=jax.ShapeDtypeStruct((B * S, DIM), jnp.float32),
        compiler_params=pltpu.CompilerParams(
            dimension_semantics=("parallel",)),
    )(out_scan.reshape(B * S, HDN), xr, Wo, Wg, bgr)

    return (y.reshape(B, S, DIM),
            memT_f.transpose(0, 1, 3, 2),
            mbufT_f.transpose(0, 1, 3, 2))
